# baseline (speedup 1.0000x reference)
import functools

import jax
import jax.numpy as jnp
from jax import lax
from jax.experimental import pallas as pl
from jax.experimental.pallas import tpu as pltpu
from jax.experimental.pallas import tpu_sc as plsc

_NUM_CORES = 2
_NUM_SUBCORES = 16
_NUM_WORKERS = _NUM_CORES * _NUM_SUBCORES

_B = 50000
_D = 128
_DW = _D // 2
_S1 = 11
_C = 64
_NCHUNK = 800
_CPW = _NCHUNK // _NUM_WORKERS
_BPW = _CPW * _C
_BPAD = 50048


def _sc_body(feat_hbm, idxt_hbm, out_hbm, idx_v, gbuf, obuf, sem0, sem1):
    wid = lax.axis_index("c") * _NUM_SUBCORES + lax.axis_index("s")
    sems = (sem0, sem1)

    base = jnp.minimum(wid * _BPW, _B - _BPW)
    for k in range(_S1):
        pltpu.sync_copy(idxt_hbm.at[pl.ds(k * _BPAD + base, _BPW)],
                        idx_v.at[pl.ds(k * _BPW, _BPW)])

    def chunk_off(t):
        row0 = jnp.minimum((wid * _CPW + t) * _C, _B - _C)
        return row0, row0 - base

    def fire(t, b):
        _, off = chunk_off(t)
        for k in range(_S1):
            idx = idx_v.at[pl.ds(k * _BPW + off, _C)]
            pltpu.async_copy(feat_hbm.at[idx], gbuf.at[b, k], sems[b])

    def drain(b):
        for k in range(_S1):
            pltpu.make_async_copy(feat_hbm.at[pl.ds(0, _C)], gbuf.at[b, k],
                                  sems[b]).wait()

    lane = lax.iota(jnp.int32, 16)

    def reduce_store(b, t):
        row0, _ = chunk_off(t)

        def srow(r, _):
            rows = jnp.full((16,), r, jnp.int32)
            for g in range(_DW // 16):
                sl = pl.ds(g * 16, 16)
                s = plsc.bitcast(gbuf[b, 0, r, sl], jnp.bfloat16)
                for k in range(1, _S1):
                    s = s + plsc.bitcast(gbuf[b, k, r, sl], jnp.bfloat16)
                obuf[r, sl] = plsc.bitcast(s, jnp.int32)
            return _

        lax.fori_loop(0, _C, srow, None)
        pltpu.sync_copy(obuf, out_hbm.at[pl.ds(row0, _C)])

    fire(0, 0)

    def pair_body(t2, _):
        t = 2 * t2
        fire(t + 1, 1)
        drain(0)
        reduce_store(0, t)
        fire(t + 2, 0)
        drain(1)
        reduce_store(1, t + 1)
        return _

    lax.fori_loop(0, (_CPW - 1) // 2, pair_body, None)
    drain(0)
    reduce_store(0, _CPW - 1)


@functools.partial(
    pl.kernel,
    out_type=jax.ShapeDtypeStruct((_B, _DW), jnp.int32),
    mesh=plsc.VectorSubcoreMesh(
        core_axis_name="c", subcore_axis_name="s",
        num_cores=_NUM_CORES, num_subcores=_NUM_SUBCORES,
    ),
    compiler_params=pltpu.CompilerParams(use_tc_tiling_on_sc=False, needs_layout_passes=False),
    scratch_types=[
        pltpu.VMEM((_S1 * _BPW,), jnp.int32),
        pltpu.VMEM((2, _S1, _C, _DW), jnp.int32),
        pltpu.VMEM((_C, _DW), jnp.int32),
        pltpu.SemaphoreType.DMA,
        pltpu.SemaphoreType.DMA,
    ],
)
def _mean_agg_sc(feat_hbm, idxt_hbm, out_hbm, idx_v, gbuf, obuf, sem0, sem1):
    _sc_body(feat_hbm, idxt_hbm, out_hbm, idx_v, gbuf, obuf, sem0, sem1)


def kernel(features, nodes, neighbours_full, num_sample):
    s = neighbours_full.shape[1]
    idxt = jnp.concatenate([neighbours_full.T, nodes[None, :]], axis=0)
    idxt = idxt + (num_sample - s)
    idxt = jnp.pad(idxt, ((0, 0), (0, _BPAD - _B))).reshape(-1)
    x = features * jnp.float32(1.0 / (s + 1))
    u = lax.bitcast_convert_type(x, jnp.uint32)
    rb = (u + jnp.uint32(0x7FFF) + ((u >> 16) & jnp.uint32(1))) >> 16
    feat32 = lax.bitcast_convert_type(
        rb[:, :_DW] | (rb[:, _DW:] << 16), jnp.int32)
    out32 = lax.bitcast_convert_type(_mean_agg_sc(feat32, idxt), jnp.uint32)
    left = lax.bitcast_convert_type(out32 << 16, jnp.float32)
    right = lax.bitcast_convert_type(out32 & jnp.uint32(0xFFFF0000),
                                     jnp.float32)
    return jnp.concatenate([left, right], axis=1)


# speedup vs baseline: 1.7891x; 1.7891x over previous
import functools

import jax
import jax.numpy as jnp
from jax import lax
from jax.experimental import pallas as pl
from jax.experimental.pallas import tpu as pltpu
from jax.experimental.pallas import tpu_sc as plsc

_NUM_CORES = 2
_NUM_SUBCORES = 16
_NUM_WORKERS = _NUM_CORES * _NUM_SUBCORES

_B = 50000
_D = 128
_S1 = 11
_C = 128
_NCHUNK = 416
_CPW = _NCHUNK // _NUM_WORKERS
_BPW = _CPW * _C
_BPAD = 50048
_INV = 1.0 / _S1


def _sc_body(feat_hbm, idxt_hbm, out_hbm, idx_v, acc, obuf,
             sem0, sem1, osem0, osem1):
    wid = lax.axis_index("c") * _NUM_SUBCORES + lax.axis_index("s")
    sems = (sem0, sem1)
    osems = (osem0, osem1)
    zeros = jnp.zeros((16,), jnp.float32)

    base = jnp.minimum(wid * _BPW, _B - _BPW)
    for k in range(_S1):
        pltpu.sync_copy(idxt_hbm.at[pl.ds(k * _BPAD + base, _BPW)],
                        idx_v.at[pl.ds(k * _BPW, _BPW)])

    def chunk_off(t):
        row0 = jnp.minimum((wid * _CPW + t) * _C, _B - _C)
        return row0, row0 - base

    def zero(b):
        def zrow(r, _):
            for j in range(_D // 16):
                acc[b, r, pl.ds(j * 16, 16)] = zeros
            return _
        lax.fori_loop(0, _C, zrow, None)

    def fire(t, b):
        _, off = chunk_off(t)
        for k in range(_S1):
            idx = idx_v.at[pl.ds(k * _BPW + off, _C)]
            pltpu.async_copy(feat_hbm.at[idx], acc.at[b], sems[b], add=True)

    def drain(b):
        for k in range(_S1):
            pltpu.make_async_copy(feat_hbm.at[pl.ds(0, _C)], acc.at[b],
                                  sems[b]).wait()

    def scale_zero_store(b, t):
        row0, _ = chunk_off(t)

        @pl.when(t >= 2)
        def _():
            pltpu.make_async_copy(obuf.at[b], out_hbm.at[pl.ds(0, _C)],
                                  osems[b]).wait()

        def srow(r, _):
            for j in range(_D // 16):
                sl = pl.ds(j * 16, 16)
                obuf[b, r, sl] = acc[b, r, sl] * _INV
                acc[b, r, sl] = zeros
            return _

        lax.fori_loop(0, _C, srow, None)
        pltpu.async_copy(obuf.at[b], out_hbm.at[pl.ds(row0, _C)], osems[b])

    zero(0)
    zero(1)
    fire(0, 0)

    def pair_body(t2, _):
        t = 2 * t2
        fire(t + 1, 1)
        drain(0)
        scale_zero_store(0, t)
        fire(t + 2, 0)
        drain(1)
        scale_zero_store(1, t + 1)
        return _

    lax.fori_loop(0, (_CPW - 1) // 2, pair_body, None)
    drain(0)
    scale_zero_store(0, _CPW - 1)
    for b in range(2):
        pltpu.make_async_copy(obuf.at[b], out_hbm.at[pl.ds(0, _C)],
                              osems[b]).wait()


@functools.partial(
    pl.kernel,
    out_type=jax.ShapeDtypeStruct((_B, _D), jnp.float32),
    mesh=plsc.VectorSubcoreMesh(
        core_axis_name="c", subcore_axis_name="s",
        num_cores=_NUM_CORES, num_subcores=_NUM_SUBCORES,
    ),
    scratch_types=[
        pltpu.VMEM((_S1 * _BPW,), jnp.int32),
        pltpu.VMEM((2, _C, _D), jnp.float32),
        pltpu.VMEM((2, _C, _D), jnp.float32),
        pltpu.SemaphoreType.DMA,
        pltpu.SemaphoreType.DMA,
        pltpu.SemaphoreType.DMA,
        pltpu.SemaphoreType.DMA,
    ],
)
def _mean_agg_sc(feat_hbm, idxt_hbm, out_hbm, idx_v, acc, obuf,
                 sem0, sem1, osem0, osem1):
    _sc_body(feat_hbm, idxt_hbm, out_hbm, idx_v, acc, obuf,
             sem0, sem1, osem0, osem1)


def kernel(features, nodes, neighbours_full, num_sample):
    s = neighbours_full.shape[1]
    idxt = jnp.concatenate([neighbours_full.T, nodes[None, :]], axis=0)
    idxt = idxt + (num_sample - s)
    idxt = jnp.pad(idxt, ((0, 0), (0, _BPAD - _B))).reshape(-1)
    return _mean_agg_sc(features, idxt)
